# Initial kernel scaffold; baseline (speedup 1.0000x reference)
import jax
import jax.numpy as jnp
from jax.experimental import pallas as pl


def kernel(pred_logits, pred_boxes, tgt_labels, tgt_boxes):
    raise NotImplementedError("write your pallas kernel here")


# speedup vs baseline: 1.5908x; 1.5908x over previous
import jax
import jax.numpy as jnp
import numpy as np
from jax.experimental import pallas as pl

_COST_CLASS = 1.0
_COST_BBOX = 5.0
_COST_GIOU = 2.0


def _cost_block_kernel(lg_ref, bx_ref, ids_ref, tbx_ref, out_ref):
    lg = lg_ref[...]
    nc = lg.shape[-1]
    nt = ids_ref.shape[-1]

    m = jnp.max(lg, axis=-1, keepdims=True)
    e = jnp.exp(lg - m)
    p = e / jnp.sum(e, axis=-1, keepdims=True)

    ids = ids_ref[...]
    cls_iota = jax.lax.broadcasted_iota(jnp.int32, (nc, nt), 0)
    onehot = (cls_iota == jnp.broadcast_to(ids, (nc, nt))).astype(jnp.float32)
    cost_class = -jax.lax.dot_general(
        p, onehot, (((1,), (0,)), ((), ())),
        preferred_element_type=jnp.float32,
    )

    cx = bx_ref[:, 0:1]
    cy = bx_ref[:, 1:2]
    w = bx_ref[:, 2:3]
    h = bx_ref[:, 3:4]
    tcx = tbx_ref[0:1, :]
    tcy = tbx_ref[1:2, :]
    tw = tbx_ref[2:3, :]
    th = tbx_ref[3:4, :]

    cost_bbox = (
        jnp.abs(cx - tcx) + jnp.abs(cy - tcy)
        + jnp.abs(w - tw) + jnp.abs(h - th)
    )

    x0 = cx - 0.5 * w
    y0 = cy - 0.5 * h
    x1 = cx + 0.5 * w
    y1 = cy + 0.5 * h
    tx0 = tcx - 0.5 * tw
    ty0 = tcy - 0.5 * th
    tx1 = tcx + 0.5 * tw
    ty1 = tcy + 0.5 * th

    area1 = (x1 - x0) * (y1 - y0)
    area2 = (tx1 - tx0) * (ty1 - ty0)
    ltx = jnp.maximum(x0, tx0)
    lty = jnp.maximum(y0, ty0)
    rbx = jnp.minimum(x1, tx1)
    rby = jnp.minimum(y1, ty1)
    iw = jnp.maximum(rbx - ltx, 0.0)
    ih = jnp.maximum(rby - lty, 0.0)
    inter = iw * ih
    union = area1 + area2 - inter
    iou = inter / union
    ltx2 = jnp.minimum(x0, tx0)
    lty2 = jnp.minimum(y0, ty0)
    rbx2 = jnp.maximum(x1, tx1)
    rby2 = jnp.maximum(y1, ty1)
    enc = jnp.maximum(rbx2 - ltx2, 0.0) * jnp.maximum(rby2 - lty2, 0.0)
    giou = iou - (enc - union) / enc
    cost_giou = -giou

    out_ref[...] = (
        _COST_BBOX * cost_bbox
        + _COST_CLASS * cost_class
        + _COST_GIOU * cost_giou
    )


def _lsa_np(cost):
    cost = np.asarray(cost, dtype=np.float64)
    transposed = False
    if cost.shape[0] > cost.shape[1]:
        cost = cost.T
        transposed = True
    n, m = cost.shape
    INF = 1e18
    u = np.zeros(n + 1)
    v = np.zeros(m + 1)
    p = np.zeros(m + 1, dtype=np.int64)
    way = np.zeros(m + 1, dtype=np.int64)
    for i in range(1, n + 1):
        p[0] = i
        j0 = 0
        minv = np.full(m + 1, INF)
        used = np.zeros(m + 1, dtype=bool)
        while True:
            used[j0] = True
            i0 = p[j0]
            cur = cost[i0 - 1, :] - u[i0] - v[1:]
            free = ~used[1:]
            better = free & (cur < minv[1:])
            idx = np.nonzero(better)[0] + 1
            minv[idx] = cur[idx - 1]
            way[idx] = j0
            cand = np.where(free, minv[1:], INF)
            j1 = int(np.argmin(cand)) + 1
            delta = cand[j1 - 1]
            u[p[used]] += delta
            v[used] -= delta
            freeidx = np.nonzero(free)[0] + 1
            minv[freeidx] -= delta
            j0 = j1
            if p[j0] == 0:
                break
        while j0 != 0:
            j1 = int(way[j0])
            p[j0] = p[j1]
            j0 = j1
    col4row = np.full(n, -1, dtype=np.int64)
    for j in range(1, m + 1):
        if p[j] != 0:
            col4row[p[j] - 1] = j - 1
    row_ind = np.arange(n, dtype=np.int64)
    col_ind = col4row
    if transposed:
        row_ind, col_ind = col_ind, row_ind
        order = np.argsort(row_ind)
        row_ind = row_ind[order]
        col_ind = col_ind[order]
    return row_ind, col_ind


def _assign_batched(Cn):
    Cn = np.asarray(Cn)
    bs, nq, total = Cn.shape
    nt = total // bs
    rows, cols = [], []
    for b in range(bs):
        r, c = _lsa_np(Cn[b, :, b * nt:(b + 1) * nt])
        rows.append(r)
        cols.append(c)
    return np.stack(rows).astype(np.int32), np.stack(cols).astype(np.int32)


def kernel(pred_logits, pred_boxes, tgt_labels, tgt_boxes):
    bs, nq, nc = pred_logits.shape
    nt_total = tgt_labels.shape[0] * tgt_labels.shape[1]
    nqf = bs * nq

    lg = pred_logits.reshape(nqf, nc)
    bx = pred_boxes.reshape(nqf, 4)
    ids = tgt_labels.reshape(1, nt_total)
    tbx = tgt_boxes.reshape(nt_total, 4).T

    bq = 1000
    while nqf % bq:
        bq //= 2
    grid = (nqf // bq,)

    cflat = pl.pallas_call(
        _cost_block_kernel,
        grid=grid,
        in_specs=[
            pl.BlockSpec((bq, nc), lambda i: (i, 0)),
            pl.BlockSpec((bq, 4), lambda i: (i, 0)),
            pl.BlockSpec((1, nt_total), lambda i: (0, 0)),
            pl.BlockSpec((4, nt_total), lambda i: (0, 0)),
        ],
        out_specs=pl.BlockSpec((bq, nt_total), lambda i: (i, 0)),
        out_shape=jax.ShapeDtypeStruct((nqf, nt_total), jnp.float32),
    )(lg, bx, ids, tbx)

    C = cflat.reshape(bs, nq, nt_total)

    k = min(nq, tgt_labels.shape[1])
    result_shapes = (
        jax.ShapeDtypeStruct((bs, k), jnp.int32),
        jax.ShapeDtypeStruct((bs, k), jnp.int32),
    )
    ind_i, ind_j = jax.pure_callback(_assign_batched, result_shapes, C)
    return (ind_i, ind_j, C)


# speedup vs baseline: 3.7002x; 2.3259x over previous
import jax
import jax.numpy as jnp
import numpy as np
from jax.experimental import pallas as pl

_COST_CLASS = 1.0
_COST_BBOX = 5.0
_COST_GIOU = 2.0


def _cost_block_kernel(lg_ref, bx_ref, ids_ref, tbx_ref, out_ref):
    lg = lg_ref[0]
    nc = lg.shape[-1]
    nt = ids_ref.shape[-1]

    m = jnp.max(lg, axis=-1, keepdims=True)
    e = jnp.exp(lg - m)
    p = e / jnp.sum(e, axis=-1, keepdims=True)

    ids = ids_ref[...]
    cls_iota = jax.lax.broadcasted_iota(jnp.int32, (nc, nt), 0)
    onehot = (cls_iota == jnp.broadcast_to(ids, (nc, nt))).astype(jnp.float32)
    cost_class = jax.lax.dot_general(
        p, onehot, (((1,), (0,)), ((), ())),
        preferred_element_type=jnp.float32,
    )

    cx = bx_ref[0, :, 0:1]
    cy = bx_ref[0, :, 1:2]
    w = bx_ref[0, :, 2:3]
    h = bx_ref[0, :, 3:4]
    tcx = tbx_ref[0:1, :]
    tcy = tbx_ref[1:2, :]
    tw = tbx_ref[2:3, :]
    th = tbx_ref[3:4, :]

    cost_bbox = (
        jnp.abs(cx - tcx) + jnp.abs(cy - tcy)
        + jnp.abs(w - tw) + jnp.abs(h - th)
    )

    x0 = cx - 0.5 * w
    y0 = cy - 0.5 * h
    x1 = cx + 0.5 * w
    y1 = cy + 0.5 * h
    tx0 = tcx - 0.5 * tw
    ty0 = tcy - 0.5 * th
    tx1 = tcx + 0.5 * tw
    ty1 = tcy + 0.5 * th

    sw = jnp.minimum(x1, tx1) - jnp.maximum(x0, tx0)
    sh = jnp.minimum(y1, ty1) - jnp.maximum(y0, ty0)
    inter = jnp.maximum(sw, 0.0) * jnp.maximum(sh, 0.0)
    enc = ((w + tw) - sw) * ((h + th) - sh)
    area1 = (x1 - x0) * (y1 - y0)
    area2 = (tx1 - tx0) * (ty1 - ty0)
    union = (area1 + area2) - inter
    giou = inter / union + (union / enc - 1.0)

    out_ref[0] = (
        _COST_BBOX * cost_bbox - cost_class - _COST_GIOU * giou
    )


def _lsa_np(cost):
    cost = np.asarray(cost, dtype=np.float64)
    transposed = False
    if cost.shape[0] > cost.shape[1]:
        cost = cost.T
        transposed = True
    n, m = cost.shape
    INF = 1e18
    u = np.zeros(n + 1)
    v = np.zeros(m + 1)
    p = np.zeros(m + 1, dtype=np.int64)
    way = np.zeros(m + 1, dtype=np.int64)
    for i in range(1, n + 1):
        p[0] = i
        j0 = 0
        minv = np.full(m + 1, INF)
        used = np.zeros(m + 1, dtype=bool)
        while True:
            used[j0] = True
            i0 = p[j0]
            cur = cost[i0 - 1, :] - u[i0] - v[1:]
            free = ~used[1:]
            better = free & (cur < minv[1:])
            idx = np.nonzero(better)[0] + 1
            minv[idx] = cur[idx - 1]
            way[idx] = j0
            cand = np.where(free, minv[1:], INF)
            j1 = int(np.argmin(cand)) + 1
            delta = cand[j1 - 1]
            u[p[used]] += delta
            v[used] -= delta
            freeidx = np.nonzero(free)[0] + 1
            minv[freeidx] -= delta
            j0 = j1
            if p[j0] == 0:
                break
        while j0 != 0:
            j1 = int(way[j0])
            p[j0] = p[j1]
            j0 = j1
    col4row = np.full(n, -1, dtype=np.int64)
    for j in range(1, m + 1):
        if p[j] != 0:
            col4row[p[j] - 1] = j - 1
    row_ind = np.arange(n, dtype=np.int64)
    col_ind = col4row
    if transposed:
        row_ind, col_ind = col_ind, row_ind
        order = np.argsort(row_ind)
        row_ind = row_ind[order]
        col_ind = col_ind[order]
    return row_ind, col_ind


def _assign_batched(Cn):
    Cn = np.asarray(Cn)
    bs, nq, total = Cn.shape
    nt = total // bs
    rows, cols = [], []
    for b in range(bs):
        r, c = _lsa_np(Cn[b, :, b * nt:(b + 1) * nt])
        rows.append(r)
        cols.append(c)
    return np.stack(rows).astype(np.int32), np.stack(cols).astype(np.int32)


def _cost_matrix_pallas(pred_logits, pred_boxes, tgt_labels, tgt_boxes):
    bs, nq, nc = pred_logits.shape
    nt = tgt_labels.shape[0] * tgt_labels.shape[1]
    ids = tgt_labels.reshape(1, nt)
    tbx = tgt_boxes.reshape(nt, 4).T

    bq = 1000
    while nq % bq:
        bq //= 2

    return pl.pallas_call(
        _cost_block_kernel,
        grid=(bs, nq // bq),
        in_specs=[
            pl.BlockSpec((1, bq, nc), lambda b, i: (b, i, 0)),
            pl.BlockSpec((1, bq, 4), lambda b, i: (b, i, 0)),
            pl.BlockSpec((1, nt), lambda b, i: (0, 0)),
            pl.BlockSpec((4, nt), lambda b, i: (0, 0)),
        ],
        out_specs=pl.BlockSpec((1, bq, nt), lambda b, i: (b, i, 0)),
        out_shape=jax.ShapeDtypeStruct((bs, nq, nt), jnp.float32),
    )(pred_logits, pred_boxes, ids, tbx)


def kernel(pred_logits, pred_boxes, tgt_labels, tgt_boxes):
    bs, nq, _ = pred_logits.shape
    C = _cost_matrix_pallas(pred_logits, pred_boxes, tgt_labels, tgt_boxes)
    k = min(nq, tgt_labels.shape[1])
    result_shapes = (
        jax.ShapeDtypeStruct((bs, k), jnp.int32),
        jax.ShapeDtypeStruct((bs, k), jnp.int32),
    )
    ind_i, ind_j = jax.pure_callback(_assign_batched, result_shapes, C)
    return (ind_i, ind_j, C)
